# baseline (speedup 1.0000x reference)
import jax
import jax.numpy as jnp
from jax.experimental import pallas as pl

_BATCH = 4
_SEQ = 4096
_D = 1024
_BS = 256


def _add_kernel(x_ref, p_ref, o_ref):
    o_ref[...] = x_ref[...] + p_ref[...][None]


def kernel(inputs, pos_table):
    return pl.pallas_call(
        _add_kernel,
        grid=(_SEQ // _BS,),
        in_specs=[
            pl.BlockSpec((_BATCH, _BS, _D), lambda s: (0, s, 0)),
            pl.BlockSpec((_BS, _D), lambda s: (s, 0)),
        ],
        out_specs=pl.BlockSpec((_BATCH, _BS, _D), lambda s: (0, s, 0)),
        out_shape=jax.ShapeDtypeStruct(inputs.shape, inputs.dtype),
    )(inputs, pos_table)
